# baseline (speedup 1.0000x reference)
import jax
import jax.numpy as jnp
from jax import lax
from jax.experimental import pallas as pl
from jax.experimental.pallas import tpu as pltpu
from jax.experimental.pallas import tpu_sc as plsc

R = 128
C = 32768
K = 3
L = 16
NC = 2
NS = 16
NW = NC * NS
RPW = R // NW
SEGC = 16
SEGW = SEGC * L
NSEG = C // SEGW

NEG_INF = float("-inf")

_GATHER_DNUMS = lax.GatherDimensionNumbers(
    offset_dims=(), collapsed_slice_dims=(0,), start_index_map=(0,))


def _dyn_gather(x, idx):
    return lax.gather(
        x, idx.reshape(L, 1), dimension_numbers=_GATHER_DNUMS,
        slice_sizes=(1,), mode=lax.GatherScatterMode.PROMISE_IN_BOUNDS)


def _insert3_lex(v, iv, m1, m2, m3, i1, i2, i3):
    gt1 = (v > m1) | ((v == m1) & (iv < i1))
    gt2 = (v > m2) | ((v == m2) & (iv < i2))
    gt3 = (v > m3) | ((v == m3) & (iv < i3))
    nm1 = jnp.where(gt1, v, m1)
    nm2 = jnp.where(gt1, m1, jnp.where(gt2, v, m2))
    nm3 = jnp.where(gt2, m2, jnp.where(gt3, v, m3))
    ni1 = jnp.where(gt1, iv, i1)
    ni2 = jnp.where(gt1, i1, jnp.where(gt2, iv, i2))
    ni3 = jnp.where(gt2, i2, jnp.where(gt3, iv, i3))
    return nm1, nm2, nm3, ni1, ni2, ni3


def _topk_body(x_hbm, vals_hbm, idx_hbm, xbuf, smax, cbuf, vout, iout,
               sem0, sem1, ssa, ssb, ssc, ssd):
    cid = lax.axis_index("c")
    sid = lax.axis_index("s")
    wid = sid * NC + cid
    base_row = wid * RPW

    lane = lax.iota(jnp.int32, L)
    neg = jnp.full((L,), NEG_INF, jnp.float32)
    zero_i = jnp.zeros((L,), jnp.int32)
    lane_base = lane * NSEG

    def phase_a(off, lo, hi, gm0):
        @plsc.parallel_loop(lo, hi, unroll=4, carry=gm0)
        def ares(sgi, gm):
            base = off + sgi * SEGW
            vs = [xbuf[pl.ds(base + q * L, L)] for q in range(SEGC)]
            while len(vs) > 1:
                vs = [jnp.maximum(vs[2 * i], vs[2 * i + 1])
                      for i in range(len(vs) // 2)]
            smax[pl.ds(sgi * L, L)] = vs[0]
            return jnp.maximum(gm, vs[0])

        return ares

    def process_row(buf, local_r, m):
        off = buf * C

        for k in range(3):
            t = m
            for s in (8, 4, 2, 1):
                t = jnp.maximum(t, _dyn_gather(t, lane ^ s))
            if k < 2:
                m = jnp.where(m == t, NEG_INF, m)
        tv = t

        @plsc.parallel_loop(0, NSEG, unroll=8, carry=zero_i)
        def bres(sgi, pv):
            sm = smax[pl.ds(sgi * L, L)]
            sel = sm >= tv
            pair_id = lane + sgi * L
            plsc.store_scatter(cbuf, [lane_base + pv], pair_id, mask=sel)
            return pv + jnp.where(sel, 1, 0)

        pv = bres

        pmax = pv
        for s in (8, 4, 2, 1):
            pmax = jnp.maximum(pmax, _dyn_gather(pmax, lane ^ s))

        def cbody(j, carry):
            m1, m2, m3, i1, i2, i3 = carry
            valid = j < pv
            pid = plsc.load_gather(
                cbuf, [lane_base + jnp.where(valid, j, 0)])
            pid = jnp.where(valid, pid, 0)
            sgi = pid >> 4
            ln = pid & (L - 1)
            ebase = sgi * SEGW + ln
            for q in range(SEGC):
                col = ebase + q * L
                v = plsc.load_gather(xbuf, [col + off])
                v = jnp.where(valid, v, NEG_INF)
                m1, m2, m3, i1, i2, i3 = _insert3_lex(
                    v, col, m1, m2, m3, i1, i2, i3)
            return (m1, m2, m3, i1, i2, i3)

        m1, m2, m3, i1, i2, i3 = lax.fori_loop(
            0, pmax[0], cbody, (neg, neg, neg, zero_i, zero_i, zero_i))

        rv = jnp.zeros((L,), jnp.float32)
        ri = jnp.zeros((L,), jnp.int32)
        for k in range(3):
            vmax, imin = m1, i1
            for s in (8, 4, 2, 1):
                perm = lane ^ s
                ov = _dyn_gather(vmax, perm)
                oi = _dyn_gather(imin, perm)
                take = (ov > vmax) | ((ov == vmax) & (oi < imin))
                vmax = jnp.where(take, ov, vmax)
                imin = jnp.where(take, oi, imin)
            win = (m1 == vmax) & (i1 == imin)
            rv = jnp.where(lane == k, vmax, rv)
            ri = jnp.where(lane == k, imin, ri)
            m1 = jnp.where(win, m2, m1)
            m2 = jnp.where(win, m3, m2)
            m3 = jnp.where(win, NEG_INF, m3)
            i1 = jnp.where(win, i2, i1)
            i2 = jnp.where(win, i3, i2)

        vout[local_r, :] = rv
        iout[local_r, :] = ri

    CSUB = C // 4
    subsems = (ssa, ssb, ssc, ssd)
    for p in range(4):
        pltpu.async_copy(x_hbm.at[base_row, pl.ds(p * CSUB, CSUB)],
                         xbuf.at[pl.ds(p * CSUB, CSUB)], subsems[p])

    def pair_body(j, carry):
        row0 = base_row + 2 * j
        pltpu.async_copy(x_hbm.at[row0 + 1], xbuf.at[pl.ds(C, C)], sem1)

        def row0_first(_):
            gm = neg
            for p in range(4):
                pltpu.make_async_copy(
                    x_hbm.at[base_row, pl.ds(p * CSUB, CSUB)],
                    xbuf.at[pl.ds(p * CSUB, CSUB)], subsems[p]).wait()
                gm = phase_a(0, p * (NSEG // 4), (p + 1) * (NSEG // 4), gm)
            return gm

        def row0_later(_):
            pltpu.make_async_copy(
                x_hbm.at[row0], xbuf.at[pl.ds(0, C)], sem0).wait()
            return phase_a(0, 0, NSEG, neg)

        m0 = lax.cond(j == 0, row0_first, row0_later, 0)
        process_row(0, 2 * j, m0)

        @pl.when(j + 1 < RPW // 2)
        def _():
            pltpu.async_copy(
                x_hbm.at[row0 + 2], xbuf.at[pl.ds(0, C)], sem0)

        pltpu.make_async_copy(
            x_hbm.at[row0 + 1], xbuf.at[pl.ds(C, C)], sem1).wait()
        m1 = phase_a(C, 0, NSEG, neg)
        process_row(1, 2 * j + 1, m1)
        return carry

    lax.fori_loop(0, RPW // 2, pair_body, 0)

    pltpu.sync_copy(vout, vals_hbm.at[pl.ds(base_row, RPW)])
    pltpu.sync_copy(iout, idx_hbm.at[pl.ds(base_row, RPW)])


@jax.jit
def kernel(x):
    mesh = plsc.VectorSubcoreMesh(core_axis_name="c", subcore_axis_name="s")
    f = pl.kernel(
        _topk_body,
        out_type=[jax.ShapeDtypeStruct((R, L), jnp.float32),
                  jax.ShapeDtypeStruct((R, L), jnp.int32)],
        mesh=mesh,
        compiler_params=pltpu.CompilerParams(needs_layout_passes=False),
        scratch_types=[
            pltpu.VMEM((2 * C,), jnp.float32),
            pltpu.VMEM((NSEG * L,), jnp.float32),
            pltpu.VMEM((L * NSEG,), jnp.int32),
            pltpu.VMEM((RPW, L), jnp.float32),
            pltpu.VMEM((RPW, L), jnp.int32),
            pltpu.SemaphoreType.DMA,
            pltpu.SemaphoreType.DMA,
            pltpu.SemaphoreType.DMA,
            pltpu.SemaphoreType.DMA,
            pltpu.SemaphoreType.DMA,
            pltpu.SemaphoreType.DMA,
        ],
    )
    vals, idx = f(x)
    return vals[:, :K], idx[:, :K]


# speedup vs baseline: 1.0404x; 1.0404x over previous
import jax
import jax.numpy as jnp
from jax import lax
from jax.experimental import pallas as pl
from jax.experimental.pallas import tpu as pltpu
from jax.experimental.pallas import tpu_sc as plsc

R = 128
C = 32768
K = 3
L = 16
NC = 2
NS = 16
NW = NC * NS
RPW = R // NW
SEGC = 16
SEGW = SEGC * L
NSEG = C // SEGW

NEG_INF = float("-inf")

_GATHER_DNUMS = lax.GatherDimensionNumbers(
    offset_dims=(), collapsed_slice_dims=(0,), start_index_map=(0,))


def _dyn_gather(x, idx):
    return lax.gather(
        x, idx.reshape(L, 1), dimension_numbers=_GATHER_DNUMS,
        slice_sizes=(1,), mode=lax.GatherScatterMode.PROMISE_IN_BOUNDS)


def _insert3_lex(v, iv, m1, m2, m3, i1, i2, i3):
    gt1 = (v > m1) | ((v == m1) & (iv < i1))
    gt2 = (v > m2) | ((v == m2) & (iv < i2))
    gt3 = (v > m3) | ((v == m3) & (iv < i3))
    nm1 = jnp.where(gt1, v, m1)
    nm2 = jnp.where(gt1, m1, jnp.where(gt2, v, m2))
    nm3 = jnp.where(gt2, m2, jnp.where(gt3, v, m3))
    ni1 = jnp.where(gt1, iv, i1)
    ni2 = jnp.where(gt1, i1, jnp.where(gt2, iv, i2))
    ni3 = jnp.where(gt2, i2, jnp.where(gt3, iv, i3))
    return nm1, nm2, nm3, ni1, ni2, ni3


def _topk_body(x_hbm, vals_hbm, idx_hbm, xbuf, smax, cbuf, vout, iout,
               sem0, sem1, ssa, ssb, ssc, ssd):
    cid = lax.axis_index("c")
    sid = lax.axis_index("s")
    wid = sid * NC + cid
    base_row = wid * RPW

    lane = lax.iota(jnp.int32, L)
    neg = jnp.full((L,), NEG_INF, jnp.float32)
    zero_i = jnp.zeros((L,), jnp.int32)
    lane_base = lane * NSEG

    def phase_a(off, lo, hi, gm0):
        @plsc.parallel_loop(lo, hi, unroll=2, carry=gm0)
        def ares(sgi, gm):
            base = off + sgi * SEGW
            vs = [xbuf[pl.ds(base + q * L, L)] for q in range(SEGC)]
            while len(vs) > 1:
                vs = [jnp.maximum(vs[2 * i], vs[2 * i + 1])
                      for i in range(len(vs) // 2)]
            smax[pl.ds(sgi * L, L)] = vs[0]
            return jnp.maximum(gm, vs[0])

        return ares

    def process_row(buf, local_r, m):
        off = buf * C

        for k in range(3):
            t = m
            for s in (8, 4, 2, 1):
                t = jnp.maximum(t, _dyn_gather(t, lane ^ s))
            if k < 2:
                m = jnp.where(m == t, NEG_INF, m)
        tv = t

        @plsc.parallel_loop(0, NSEG, unroll=4, carry=zero_i)
        def bres(sgi, pv):
            sm = smax[pl.ds(sgi * L, L)]
            sel = sm >= tv
            pair_id = lane + sgi * L
            plsc.store_scatter(cbuf, [lane_base + pv], pair_id, mask=sel)
            return pv + jnp.where(sel, 1, 0)

        pv = bres

        pmax = pv
        for s in (8, 4, 2, 1):
            pmax = jnp.maximum(pmax, _dyn_gather(pmax, lane ^ s))

        def cbody(j, carry):
            m1, m2, m3, i1, i2, i3 = carry
            valid = j < pv
            pid = plsc.load_gather(
                cbuf, [lane_base + jnp.where(valid, j, 0)])
            pid = jnp.where(valid, pid, 0)
            sgi = pid >> 4
            ln = pid & (L - 1)
            ebase = sgi * SEGW + ln
            for q in range(SEGC):
                col = ebase + q * L
                v = plsc.load_gather(xbuf, [col + off])
                v = jnp.where(valid, v, NEG_INF)
                m1, m2, m3, i1, i2, i3 = _insert3_lex(
                    v, col, m1, m2, m3, i1, i2, i3)
            return (m1, m2, m3, i1, i2, i3)

        m1, m2, m3, i1, i2, i3 = lax.fori_loop(
            0, pmax[0], cbody, (neg, neg, neg, zero_i, zero_i, zero_i))

        rv = jnp.zeros((L,), jnp.float32)
        ri = jnp.zeros((L,), jnp.int32)
        for k in range(3):
            vmax, imin = m1, i1
            for s in (8, 4, 2, 1):
                perm = lane ^ s
                ov = _dyn_gather(vmax, perm)
                oi = _dyn_gather(imin, perm)
                take = (ov > vmax) | ((ov == vmax) & (oi < imin))
                vmax = jnp.where(take, ov, vmax)
                imin = jnp.where(take, oi, imin)
            win = (m1 == vmax) & (i1 == imin)
            rv = jnp.where(lane == k, vmax, rv)
            ri = jnp.where(lane == k, imin, ri)
            m1 = jnp.where(win, m2, m1)
            m2 = jnp.where(win, m3, m2)
            m3 = jnp.where(win, NEG_INF, m3)
            i1 = jnp.where(win, i2, i1)
            i2 = jnp.where(win, i3, i2)

        vout[local_r, :] = rv
        iout[local_r, :] = ri

    CSUB = C // 4
    subsems = (ssa, ssb, ssc, ssd)
    for p in range(4):
        pltpu.async_copy(x_hbm.at[base_row, pl.ds(p * CSUB, CSUB)],
                         xbuf.at[pl.ds(p * CSUB, CSUB)], subsems[p])

    def pair_body(j, carry):
        row0 = base_row + 2 * j
        pltpu.async_copy(x_hbm.at[row0 + 1], xbuf.at[pl.ds(C, C)], sem1)

        def row0_first(_):
            gm = neg
            for p in range(4):
                pltpu.make_async_copy(
                    x_hbm.at[base_row, pl.ds(p * CSUB, CSUB)],
                    xbuf.at[pl.ds(p * CSUB, CSUB)], subsems[p]).wait()
                gm = phase_a(0, p * (NSEG // 4), (p + 1) * (NSEG // 4), gm)
            return gm

        def row0_later(_):
            pltpu.make_async_copy(
                x_hbm.at[row0], xbuf.at[pl.ds(0, C)], sem0).wait()
            return phase_a(0, 0, NSEG, neg)

        m0 = lax.cond(j == 0, row0_first, row0_later, 0)
        process_row(0, 2 * j, m0)

        @pl.when(j + 1 < RPW // 2)
        def _():
            pltpu.async_copy(
                x_hbm.at[row0 + 2], xbuf.at[pl.ds(0, C)], sem0)

        pltpu.make_async_copy(
            x_hbm.at[row0 + 1], xbuf.at[pl.ds(C, C)], sem1).wait()
        m1 = phase_a(C, 0, NSEG, neg)
        process_row(1, 2 * j + 1, m1)
        return carry

    lax.fori_loop(0, RPW // 2, pair_body, 0)

    pltpu.sync_copy(vout, vals_hbm.at[pl.ds(base_row, RPW)])
    pltpu.sync_copy(iout, idx_hbm.at[pl.ds(base_row, RPW)])


@jax.jit
def kernel(x):
    mesh = plsc.VectorSubcoreMesh(core_axis_name="c", subcore_axis_name="s")
    f = pl.kernel(
        _topk_body,
        out_type=[jax.ShapeDtypeStruct((R, L), jnp.float32),
                  jax.ShapeDtypeStruct((R, L), jnp.int32)],
        mesh=mesh,
        compiler_params=pltpu.CompilerParams(needs_layout_passes=False),
        scratch_types=[
            pltpu.VMEM((2 * C,), jnp.float32),
            pltpu.VMEM((NSEG * L,), jnp.float32),
            pltpu.VMEM((L * NSEG,), jnp.int32),
            pltpu.VMEM((RPW, L), jnp.float32),
            pltpu.VMEM((RPW, L), jnp.int32),
            pltpu.SemaphoreType.DMA,
            pltpu.SemaphoreType.DMA,
            pltpu.SemaphoreType.DMA,
            pltpu.SemaphoreType.DMA,
            pltpu.SemaphoreType.DMA,
            pltpu.SemaphoreType.DMA,
        ],
    )
    vals, idx = f(x)
    return vals[:, :K], idx[:, :K]
